# baseline (speedup 1.0000x reference)
import jax
import jax.numpy as jnp
from jax.experimental import pallas as pl


def _body(x_ref, a_ref, m_ref, o_ref):
    m = m_ref[...]
    o_ref[...] = x_ref[...] + jnp.where(m[:, :, None] != 0.0, a_ref[...], 0.0)


def kernel(x, attack, attack_mask):
    B, S, D = x.shape
    BS = 256
    m = attack_mask.astype(jnp.float32)
    return pl.pallas_call(
        _body,
        grid=(S // BS,),
        in_specs=[
            pl.BlockSpec((B, BS, D), lambda s: (0, s, 0)),
            pl.BlockSpec((B, BS, D), lambda s: (0, s, 0)),
            pl.BlockSpec((B, BS), lambda s: (0, s)),
        ],
        out_specs=pl.BlockSpec((B, BS, D), lambda s: (0, s, 0)),
        out_shape=jax.ShapeDtypeStruct(x.shape, x.dtype),
    )(x, attack, m)
